# baseline (speedup 1.0000x reference)
import jax
import jax.numpy as jnp
from jax import lax
from jax.experimental import pallas as pl
from jax.experimental.pallas import tpu as pltpu
from jax.experimental.pallas import tpu_sc as plsc

BATCH = 16384
LENGTH = 1024
NBLK = 8
NW = 32
RPW = BATCH // NW
CH = 8
NCH = RPW // CH


def _sc_body(z_hbm, cr_hbm, ci_hbm, o_hbm, xb, ob, crv, civ,
             isem0, isem1, osem0, osem1):
    wid = lax.axis_index("s") * 2 + lax.axis_index("c")
    base_row = wid * RPW

    pltpu.sync_copy(cr_hbm, crv)
    pltpu.sync_copy(ci_hbm, civ)

    lane = lax.iota(jnp.int32, 16)
    m4 = lane & 3
    cpat = lane + jnp.where(m4 == 1, 1, 0) - jnp.where(m4 == 2, 1, 0)

    isems = (isem0, isem1)
    osems = (osem0, osem1)

    def in_copy(c, par):
        row0 = base_row + c * CH
        return pltpu.make_async_copy(
            z_hbm.at[pl.ds(row0, CH)], xb.at[par], isems[par])

    def out_copy(c, par):
        row0 = base_row + c * CH
        return pltpu.make_async_copy(
            ob.at[par], o_hbm.at[pl.ds(row0, CH)], osems[par])

    in_copy(0, 0).start()
    in_copy(1, 1).start()

    def chunk_pair(c2, carry):
        for par in (0, 1):
            c = 2 * c2 + par
            in_copy(c, par).wait()

            @pl.when(c >= 2)
            def _():
                out_copy(c - 2, par).wait()

            xbp = xb.at[par]
            obp = ob.at[par]

            for k in range(NBLK):
                for g in range(8):
                    c0 = g * 16
                    a = crv[pl.ds(k * 128 + c0, 16)]
                    b = civ[pl.ds(k * 128 + c0, 16)]
                    for r in range(CH):
                        xre = xbp[r, 2 * k, pl.ds(c0, 16)]
                        xim = xbp[r, 2 * k + 1, pl.ds(c0, 16)]
                        yre = xre.at[cpat].get(mode="promise_in_bounds")
                        yim = xim.at[cpat].get(mode="promise_in_bounds")
                        ore = a * yre - b * yim
                        oim = a * yim + b * yre
                        obp[r, 2 * k, pl.ds(c0, 16)] = ore
                        obp[r, 2 * k + 1, pl.ds(c0, 16)] = oim

            out_copy(c, par).start()

            @pl.when(c + 2 < NCH)
            def _():
                in_copy(c + 2, par).start()
        return carry

    lax.fori_loop(0, NCH // 2, chunk_pair, 0, unroll=False)

    out_copy(NCH - 2, 0).wait()
    out_copy(NCH - 1, 1).wait()


def kernel(x, forward_indices, crossings):
    del forward_indices
    z = x.reshape(BATCH, NBLK, 128, 2).transpose(0, 1, 3, 2).reshape(
        BATCH, 2 * NBLK, 128)
    cr = crossings[:, 0]
    ci = crossings[:, 1]

    run = pl.kernel(
        _sc_body,
        out_type=jax.ShapeDtypeStruct((BATCH, 2 * NBLK, 128), jnp.float32),
        mesh=plsc.VectorSubcoreMesh(core_axis_name="c", subcore_axis_name="s"),
        compiler_params=pltpu.CompilerParams(
            use_tc_tiling_on_sc=False, needs_layout_passes=False),
        scratch_types=[
            pltpu.VMEM((2, CH, 2 * NBLK, 128), jnp.float32),
            pltpu.VMEM((2, CH, 2 * NBLK, 128), jnp.float32),
            pltpu.VMEM((LENGTH,), jnp.float32),
            pltpu.VMEM((LENGTH,), jnp.float32),
            pltpu.SemaphoreType.DMA,
            pltpu.SemaphoreType.DMA,
            pltpu.SemaphoreType.DMA,
            pltpu.SemaphoreType.DMA,
        ],
    )
    oz = run(z, cr, ci)
    return oz.reshape(BATCH, NBLK, 2, 128).transpose(0, 1, 3, 2).reshape(
        BATCH, LENGTH, 2)


# speedup vs baseline: 1.6554x; 1.6554x over previous
import jax
import jax.numpy as jnp
from jax import lax
from jax.experimental import pallas as pl
from jax.experimental.pallas import tpu as pltpu
from jax.experimental.pallas import tpu_sc as plsc

BATCH = 16384
LENGTH = 1024
NBLK = 8
NW = 32
RPW = BATCH // NW
CH = 8
NCH = RPW // CH


def _sc_body(z_hbm, cr_hbm, ci_hbm, o_hbm, xb, ob, crv, civ,
             isem0, isem1, osem0, osem1):
    wid = lax.axis_index("s") * 2 + lax.axis_index("c")
    base_row = wid * RPW

    pltpu.sync_copy(cr_hbm, crv)
    pltpu.sync_copy(ci_hbm, civ)

    lane = lax.iota(jnp.int32, 16)
    m4 = lane & 3
    cpat = lane + jnp.where(m4 == 1, 1, 0) - jnp.where(m4 == 2, 1, 0)

    isems = (isem0, isem1)
    osems = (osem0, osem1)

    def in_copy(c, par):
        row0 = base_row + c * CH
        return pltpu.make_async_copy(
            z_hbm.at[pl.ds(row0, CH)], xb.at[par], isems[par])

    def out_copy(c, par):
        row0 = base_row + c * CH
        return pltpu.make_async_copy(
            ob.at[par], o_hbm.at[pl.ds(row0, CH)], osems[par])

    in_copy(0, 0).start()
    in_copy(1, 1).start()

    def chunk_pair(c2, carry):
        for par in (0, 1):
            c = 2 * c2 + par
            in_copy(c, par).wait()

            @pl.when(c >= 2)
            def _():
                out_copy(c - 2, par).wait()

            xbp = xb.at[par]
            obp = ob.at[par]

            @plsc.parallel_loop(0, NBLK)
            def blk_body(k):
                for g in range(8):
                    c0 = g * 16
                    a = crv[pl.ds(k * 128 + c0, 16)]
                    b = civ[pl.ds(k * 128 + c0, 16)]
                    for r in range(CH):
                        xre = xbp[r, 2 * k, pl.ds(c0, 16)]
                        xim = xbp[r, 2 * k + 1, pl.ds(c0, 16)]
                        yre = xre.at[cpat].get(mode="promise_in_bounds")
                        yim = xim.at[cpat].get(mode="promise_in_bounds")
                        ore = a * yre - b * yim
                        oim = a * yim + b * yre
                        obp[r, 2 * k, pl.ds(c0, 16)] = ore
                        obp[r, 2 * k + 1, pl.ds(c0, 16)] = oim

            out_copy(c, par).start()

            @pl.when(c + 2 < NCH)
            def _():
                in_copy(c + 2, par).start()
        return carry

    lax.fori_loop(0, NCH // 2, chunk_pair, 0, unroll=False)

    out_copy(NCH - 2, 0).wait()
    out_copy(NCH - 1, 1).wait()


def kernel(x, forward_indices, crossings):
    del forward_indices
    z = x.reshape(BATCH, NBLK, 128, 2).transpose(0, 1, 3, 2).reshape(
        BATCH, 2 * NBLK, 128)
    cr = crossings[:, 0]
    ci = crossings[:, 1]

    run = pl.kernel(
        _sc_body,
        out_type=jax.ShapeDtypeStruct((BATCH, 2 * NBLK, 128), jnp.float32),
        mesh=plsc.VectorSubcoreMesh(core_axis_name="c", subcore_axis_name="s"),
        compiler_params=pltpu.CompilerParams(
            use_tc_tiling_on_sc=False, needs_layout_passes=False),
        scratch_types=[
            pltpu.VMEM((2, CH, 2 * NBLK, 128), jnp.float32),
            pltpu.VMEM((2, CH, 2 * NBLK, 128), jnp.float32),
            pltpu.VMEM((LENGTH,), jnp.float32),
            pltpu.VMEM((LENGTH,), jnp.float32),
            pltpu.SemaphoreType.DMA,
            pltpu.SemaphoreType.DMA,
            pltpu.SemaphoreType.DMA,
            pltpu.SemaphoreType.DMA,
        ],
    )
    oz = run(z, cr, ci)
    return oz.reshape(BATCH, NBLK, 2, 128).transpose(0, 1, 3, 2).reshape(
        BATCH, LENGTH, 2)


# speedup vs baseline: 1.7420x; 1.0523x over previous
import jax
import jax.numpy as jnp
from jax import lax
from jax.experimental import pallas as pl
from jax.experimental.pallas import tpu as pltpu
from jax.experimental.pallas import tpu_sc as plsc

BATCH = 16384
LENGTH = 1024
NBLK = 8
NW = 32
RPW = BATCH // NW
CH = 4
NCH = RPW // CH


def _sc_body(z_hbm, cr_hbm, ci_hbm, o_hbm, xb, ob, crv, civ,
             isem0, isem1, isem2, isem3, osem0, osem1, osem2, osem3):
    wid = lax.axis_index("s") * 2 + lax.axis_index("c")
    base_row = wid * RPW

    pltpu.sync_copy(cr_hbm, crv)
    pltpu.sync_copy(ci_hbm, civ)

    lane = lax.iota(jnp.int32, 16)
    m4 = lane & 3
    cpat = lane + jnp.where(m4 == 1, 1, 0) - jnp.where(m4 == 2, 1, 0)

    isems = (isem0, isem1, isem2, isem3)
    osems = (osem0, osem1, osem2, osem3)

    def in_copy(c, par):
        row0 = base_row + c * CH
        return pltpu.make_async_copy(
            z_hbm.at[pl.ds(row0, CH)], xb.at[par], isems[par])

    def out_copy(c, par):
        row0 = base_row + c * CH
        return pltpu.make_async_copy(
            ob.at[par], o_hbm.at[pl.ds(row0, CH)], osems[par])

    in_copy(0, 0).start()
    in_copy(1, 1).start()
    in_copy(2, 2).start()
    in_copy(3, 3).start()

    def chunk_pair(c2, carry):
        for par in (0, 1, 2, 3):
            c = 4 * c2 + par
            in_copy(c, par).wait()

            @pl.when(c >= 4)
            def _():
                out_copy(c - 4, par).wait()

            xbp = xb.at[par]
            obp = ob.at[par]

            @plsc.parallel_loop(0, NBLK)
            def blk_body(k):
                for g in range(8):
                    c0 = g * 16
                    a = crv[pl.ds(k * 128 + c0, 16)]
                    b = civ[pl.ds(k * 128 + c0, 16)]
                    for r in range(CH):
                        xre = xbp[r, 2 * k, pl.ds(c0, 16)]
                        xim = xbp[r, 2 * k + 1, pl.ds(c0, 16)]
                        yre = xre.at[cpat].get(mode="promise_in_bounds")
                        yim = xim.at[cpat].get(mode="promise_in_bounds")
                        ore = a * yre - b * yim
                        oim = a * yim + b * yre
                        obp[r, 2 * k, pl.ds(c0, 16)] = ore
                        obp[r, 2 * k + 1, pl.ds(c0, 16)] = oim

            out_copy(c, par).start()

            @pl.when(c + 4 < NCH)
            def _():
                in_copy(c + 4, par).start()
        return carry

    lax.fori_loop(0, NCH // 4, chunk_pair, 0, unroll=False)

    out_copy(NCH - 4, 0).wait()
    out_copy(NCH - 3, 1).wait()
    out_copy(NCH - 2, 2).wait()
    out_copy(NCH - 1, 3).wait()


def kernel(x, forward_indices, crossings):
    del forward_indices
    z = x.reshape(BATCH, NBLK, 128, 2).transpose(0, 1, 3, 2).reshape(
        BATCH, 2 * NBLK, 128)
    cr = crossings[:, 0]
    ci = crossings[:, 1]

    run = pl.kernel(
        _sc_body,
        out_type=jax.ShapeDtypeStruct((BATCH, 2 * NBLK, 128), jnp.float32),
        mesh=plsc.VectorSubcoreMesh(core_axis_name="c", subcore_axis_name="s"),
        compiler_params=pltpu.CompilerParams(
            use_tc_tiling_on_sc=False, needs_layout_passes=False),
        scratch_types=[
            pltpu.VMEM((4, CH, 2 * NBLK, 128), jnp.float32),
            pltpu.VMEM((4, CH, 2 * NBLK, 128), jnp.float32),
            pltpu.VMEM((LENGTH,), jnp.float32),
            pltpu.VMEM((LENGTH,), jnp.float32),
            pltpu.SemaphoreType.DMA,
            pltpu.SemaphoreType.DMA,
            pltpu.SemaphoreType.DMA,
            pltpu.SemaphoreType.DMA,
            pltpu.SemaphoreType.DMA,
            pltpu.SemaphoreType.DMA,
            pltpu.SemaphoreType.DMA,
            pltpu.SemaphoreType.DMA,
        ],
    )
    oz = run(z, cr, ci)
    return oz.reshape(BATCH, NBLK, 2, 128).transpose(0, 1, 3, 2).reshape(
        BATCH, LENGTH, 2)


# speedup vs baseline: 2.0608x; 1.1830x over previous
import jax
import jax.numpy as jnp
from jax import lax
from jax.experimental import pallas as pl
from jax.experimental.pallas import tpu as pltpu
from jax.experimental.pallas import tpu_sc as plsc

BATCH = 16384
LENGTH = 1024
NBLK = 8
NW = 32
RPW = BATCH // NW
CH = 8
NCH = RPW // CH


def _sc_body(z_hbm, cr_hbm, ci_hbm, o_hbm, xb, crv, civ,
             isem0, isem1, isem2, isem3, osem0, osem1, osem2, osem3):
    wid = lax.axis_index("s") * 2 + lax.axis_index("c")
    base_row = wid * RPW

    pltpu.sync_copy(cr_hbm, crv)
    pltpu.sync_copy(ci_hbm, civ)

    lane = lax.iota(jnp.int32, 16)
    m4 = lane & 3
    cpat = lane + jnp.where(m4 == 1, 1, 0) - jnp.where(m4 == 2, 1, 0)

    isems = (isem0, isem1, isem2, isem3)
    osems = (osem0, osem1, osem2, osem3)

    def in_copy(c, par):
        row0 = base_row + c * CH
        return pltpu.make_async_copy(
            z_hbm.at[pl.ds(row0, CH)], xb.at[par], isems[par])

    def out_copy(c, par):
        row0 = base_row + c * CH
        return pltpu.make_async_copy(
            xb.at[par], o_hbm.at[pl.ds(row0, CH)], osems[par])

    in_copy(0, 0).start()
    in_copy(1, 1).start()

    def chunk_pair(c2, carry):
        for par in (0, 1, 2, 3):
            c = 4 * c2 + par
            in_copy(c, par).wait()

            xbp = xb.at[par]
            obp = xb.at[par]

            @plsc.parallel_loop(0, NBLK)
            def blk_body(k):
                for g in range(8):
                    c0 = g * 16
                    a = crv[pl.ds(k * 128 + c0, 16)]
                    b = civ[pl.ds(k * 128 + c0, 16)]
                    for r in range(CH):
                        xre = xbp[r, 2 * k, pl.ds(c0, 16)]
                        xim = xbp[r, 2 * k + 1, pl.ds(c0, 16)]
                        yre = xre.at[cpat].get(mode="promise_in_bounds")
                        yim = xim.at[cpat].get(mode="promise_in_bounds")
                        ore = a * yre - b * yim
                        oim = a * yim + b * yre
                        obp[r, 2 * k, pl.ds(c0, 16)] = ore
                        obp[r, 2 * k + 1, pl.ds(c0, 16)] = oim

            out_copy(c, par).start()

            par2 = (par + 2) % 4

            @pl.when(c + 2 < NCH)
            def _():
                @pl.when(c >= 2)
                def _():
                    out_copy(c - 2, par2).wait()

                in_copy(c + 2, par2).start()
        return carry

    lax.fori_loop(0, NCH // 4, chunk_pair, 0, unroll=False)

    out_copy(NCH - 2, (NCH - 2) % 4).wait()
    out_copy(NCH - 1, (NCH - 1) % 4).wait()


def kernel(x, forward_indices, crossings):
    del forward_indices
    z = x.reshape(BATCH, NBLK, 128, 2).transpose(0, 1, 3, 2).reshape(
        BATCH, 2 * NBLK, 128)
    cr = crossings[:, 0]
    ci = crossings[:, 1]

    run = pl.kernel(
        _sc_body,
        out_type=jax.ShapeDtypeStruct((BATCH, 2 * NBLK, 128), jnp.float32),
        mesh=plsc.VectorSubcoreMesh(core_axis_name="c", subcore_axis_name="s"),
        compiler_params=pltpu.CompilerParams(
            use_tc_tiling_on_sc=False, needs_layout_passes=False),
        scratch_types=[
            pltpu.VMEM((4, CH, 2 * NBLK, 128), jnp.float32),
            pltpu.VMEM((LENGTH,), jnp.float32),
            pltpu.VMEM((LENGTH,), jnp.float32),
            pltpu.SemaphoreType.DMA,
            pltpu.SemaphoreType.DMA,
            pltpu.SemaphoreType.DMA,
            pltpu.SemaphoreType.DMA,
            pltpu.SemaphoreType.DMA,
            pltpu.SemaphoreType.DMA,
            pltpu.SemaphoreType.DMA,
            pltpu.SemaphoreType.DMA,
        ],
    )
    oz = run(z, cr, ci)
    return oz.reshape(BATCH, NBLK, 2, 128).transpose(0, 1, 3, 2).reshape(
        BATCH, LENGTH, 2)


# speedup vs baseline: 2.2630x; 1.0982x over previous
import jax
import jax.numpy as jnp
from jax import lax
from jax.experimental import pallas as pl
from jax.experimental.pallas import tpu as pltpu
from jax.experimental.pallas import tpu_sc as plsc

BATCH = 16384
LENGTH = 1024
NBLK = 8
NW = 32
RPW = BATCH // NW
CH = 4
NCH = RPW // CH


def _sc_body(z_hbm, cr_hbm, ci_hbm, o_hbm, xb, crv, civ,
             isem0, isem1, isem2, isem3, isem4, isem5, isem6, isem7,
             osem0, osem1, osem2, osem3, osem4, osem5, osem6, osem7):
    wid = lax.axis_index("s") * 2 + lax.axis_index("c")
    base_row = wid * RPW

    pltpu.sync_copy(cr_hbm, crv)
    pltpu.sync_copy(ci_hbm, civ)

    lane = lax.iota(jnp.int32, 16)
    m4 = lane & 3
    cpat = lane + jnp.where(m4 == 1, 1, 0) - jnp.where(m4 == 2, 1, 0)

    isems = (isem0, isem1, isem2, isem3, isem4, isem5, isem6, isem7)
    osems = (osem0, osem1, osem2, osem3, osem4, osem5, osem6, osem7)

    def in_copy(c, par):
        row0 = base_row + c * CH
        return pltpu.make_async_copy(
            z_hbm.at[pl.ds(row0, CH)], xb.at[par], isems[par])

    def out_copy(c, par):
        row0 = base_row + c * CH
        return pltpu.make_async_copy(
            xb.at[par], o_hbm.at[pl.ds(row0, CH)], osems[par])

    in_copy(0, 0).start()
    in_copy(1, 1).start()
    in_copy(2, 2).start()
    in_copy(3, 3).start()

    def chunk_pair(c2, carry):
        for par in (0, 1, 2, 3, 4, 5, 6, 7):
            c = 8 * c2 + par
            in_copy(c, par).wait()

            xbp = xb.at[par]
            obp = xb.at[par]

            @plsc.parallel_loop(0, NBLK)
            def blk_body(k):
                for g in range(8):
                    c0 = g * 16
                    a = crv[pl.ds(k * 128 + c0, 16)]
                    b = civ[pl.ds(k * 128 + c0, 16)]
                    for r in range(CH):
                        xre = xbp[r, 2 * k, pl.ds(c0, 16)]
                        xim = xbp[r, 2 * k + 1, pl.ds(c0, 16)]
                        yre = xre.at[cpat].get(mode="promise_in_bounds")
                        yim = xim.at[cpat].get(mode="promise_in_bounds")
                        ore = a * yre - b * yim
                        oim = a * yim + b * yre
                        obp[r, 2 * k, pl.ds(c0, 16)] = ore
                        obp[r, 2 * k + 1, pl.ds(c0, 16)] = oim

            out_copy(c, par).start()

            par2 = (par + 4) % 8

            @pl.when(c + 4 < NCH)
            def _():
                @pl.when(c >= 4)
                def _():
                    out_copy(c - 4, par2).wait()

                in_copy(c + 4, par2).start()
        return carry

    lax.fori_loop(0, NCH // 8, chunk_pair, 0, unroll=False)

    out_copy(NCH - 4, (NCH - 4) % 8).wait()
    out_copy(NCH - 3, (NCH - 3) % 8).wait()
    out_copy(NCH - 2, (NCH - 2) % 8).wait()
    out_copy(NCH - 1, (NCH - 1) % 8).wait()


def kernel(x, forward_indices, crossings):
    del forward_indices
    z = x.reshape(BATCH, NBLK, 128, 2).transpose(0, 1, 3, 2).reshape(
        BATCH, 2 * NBLK, 128)
    cr = crossings[:, 0]
    ci = crossings[:, 1]

    run = pl.kernel(
        _sc_body,
        out_type=jax.ShapeDtypeStruct((BATCH, 2 * NBLK, 128), jnp.float32),
        mesh=plsc.VectorSubcoreMesh(core_axis_name="c", subcore_axis_name="s"),
        compiler_params=pltpu.CompilerParams(
            use_tc_tiling_on_sc=False, needs_layout_passes=False),
        scratch_types=[
            pltpu.VMEM((8, CH, 2 * NBLK, 128), jnp.float32),
            pltpu.VMEM((LENGTH,), jnp.float32),
            pltpu.VMEM((LENGTH,), jnp.float32),
            pltpu.SemaphoreType.DMA,
            pltpu.SemaphoreType.DMA,
            pltpu.SemaphoreType.DMA,
            pltpu.SemaphoreType.DMA,
            pltpu.SemaphoreType.DMA,
            pltpu.SemaphoreType.DMA,
            pltpu.SemaphoreType.DMA,
            pltpu.SemaphoreType.DMA,
            pltpu.SemaphoreType.DMA,
            pltpu.SemaphoreType.DMA,
            pltpu.SemaphoreType.DMA,
            pltpu.SemaphoreType.DMA,
            pltpu.SemaphoreType.DMA,
            pltpu.SemaphoreType.DMA,
            pltpu.SemaphoreType.DMA,
            pltpu.SemaphoreType.DMA,
        ],
    )
    oz = run(z, cr, ci)
    return oz.reshape(BATCH, NBLK, 2, 128).transpose(0, 1, 3, 2).reshape(
        BATCH, LENGTH, 2)


# speedup vs baseline: 2.3260x; 1.0278x over previous
import jax
import jax.numpy as jnp
from jax import lax
from jax.experimental import pallas as pl
from jax.experimental.pallas import tpu as pltpu
from jax.experimental.pallas import tpu_sc as plsc

BATCH = 16384
LENGTH = 1024
NBLK = 8
NW = 32
RPW = BATCH // NW
CH = 4
NCH = RPW // CH


def _sc_body(z_hbm, cr_hbm, ci_hbm, o_hbm, xb, crv, civ,
             isem0, isem1, isem2, isem3, isem4, isem5, isem6, isem7,
             osem0, osem1, osem2, osem3, osem4, osem5, osem6, osem7):
    wid = lax.axis_index("s") * 2 + lax.axis_index("c")
    base_row = wid * RPW

    pltpu.sync_copy(cr_hbm, crv)
    pltpu.sync_copy(ci_hbm, civ)

    lane = lax.iota(jnp.int32, 16)
    m4 = lane & 3
    cpat = lane + jnp.where(m4 == 1, 1, 0) - jnp.where(m4 == 2, 1, 0)

    isems = (isem0, isem1, isem2, isem3, isem4, isem5, isem6, isem7)
    osems = (osem0, osem1, osem2, osem3, osem4, osem5, osem6, osem7)

    def in_copy(c, par):
        row0 = base_row + c * CH
        return pltpu.make_async_copy(
            z_hbm.at[pl.ds(row0, CH)], xb.at[par], isems[par])

    def out_copy(c, par):
        row0 = base_row + c * CH
        return pltpu.make_async_copy(
            xb.at[par], o_hbm.at[pl.ds(row0, CH)], osems[par])

    in_copy(0, 0).start()
    in_copy(1, 1).start()
    in_copy(2, 2).start()
    in_copy(3, 3).start()
    in_copy(4, 4).start()
    in_copy(5, 5).start()

    def chunk_pair(c2, carry):
        for par in (0, 1, 2, 3, 4, 5, 6, 7):
            c = 8 * c2 + par
            in_copy(c, par).wait()

            xbp = xb.at[par]
            obp = xb.at[par]

            @plsc.parallel_loop(0, NBLK)
            def blk_body(k):
                for g in range(8):
                    c0 = g * 16
                    a = crv[pl.ds(k * 128 + c0, 16)]
                    b = civ[pl.ds(k * 128 + c0, 16)]
                    for r in range(CH):
                        xre = xbp[r, 2 * k, pl.ds(c0, 16)]
                        xim = xbp[r, 2 * k + 1, pl.ds(c0, 16)]
                        yre = xre.at[cpat].get(mode="promise_in_bounds")
                        yim = xim.at[cpat].get(mode="promise_in_bounds")
                        ore = a * yre - b * yim
                        oim = a * yim + b * yre
                        obp[r, 2 * k, pl.ds(c0, 16)] = ore
                        obp[r, 2 * k + 1, pl.ds(c0, 16)] = oim

            out_copy(c, par).start()

            par2 = (par + 6) % 8

            @pl.when(c + 6 < NCH)
            def _():
                @pl.when(c >= 2)
                def _():
                    out_copy(c - 2, par2).wait()

                in_copy(c + 6, par2).start()
        return carry

    lax.fori_loop(0, NCH // 8, chunk_pair, 0, unroll=False)

    for d in range(NCH - 8, NCH):
        out_copy(d, d % 8).wait()


def kernel(x, forward_indices, crossings):
    del forward_indices
    z = x.reshape(BATCH, NBLK, 128, 2).transpose(0, 1, 3, 2).reshape(
        BATCH, 2 * NBLK, 128)
    cr = crossings[:, 0]
    ci = crossings[:, 1]

    run = pl.kernel(
        _sc_body,
        out_type=jax.ShapeDtypeStruct((BATCH, 2 * NBLK, 128), jnp.float32),
        mesh=plsc.VectorSubcoreMesh(core_axis_name="c", subcore_axis_name="s"),
        compiler_params=pltpu.CompilerParams(
            use_tc_tiling_on_sc=False, needs_layout_passes=False),
        scratch_types=[
            pltpu.VMEM((8, CH, 2 * NBLK, 128), jnp.float32),
            pltpu.VMEM((LENGTH,), jnp.float32),
            pltpu.VMEM((LENGTH,), jnp.float32),
            pltpu.SemaphoreType.DMA,
            pltpu.SemaphoreType.DMA,
            pltpu.SemaphoreType.DMA,
            pltpu.SemaphoreType.DMA,
            pltpu.SemaphoreType.DMA,
            pltpu.SemaphoreType.DMA,
            pltpu.SemaphoreType.DMA,
            pltpu.SemaphoreType.DMA,
            pltpu.SemaphoreType.DMA,
            pltpu.SemaphoreType.DMA,
            pltpu.SemaphoreType.DMA,
            pltpu.SemaphoreType.DMA,
            pltpu.SemaphoreType.DMA,
            pltpu.SemaphoreType.DMA,
            pltpu.SemaphoreType.DMA,
            pltpu.SemaphoreType.DMA,
        ],
    )
    oz = run(z, cr, ci)
    return oz.reshape(BATCH, NBLK, 2, 128).transpose(0, 1, 3, 2).reshape(
        BATCH, LENGTH, 2)
